# baseline (speedup 1.0000x reference)
import functools

import jax
import jax.numpy as jnp
from jax import lax
from jax.experimental import pallas as pl
from jax.experimental.pallas import tpu as pltpu
from jax.experimental.pallas import tpu_sc as plsc

DS_SIZE = 100000
DIM = 64
BATCH = 1024

_info = plsc.get_sparse_core_info()
_NC, _NS, _L = _info.num_cores, _info.num_subcores, _info.num_lanes
_NGRP = 8
_GF = DIM // _NGRP
_CPT = 8
_W = 3200
_PAD_COLS = 100096
_TAIL_OFF = _PAD_COLS - _W
_NBUF = 3


def _gather_body(memt_hbm, idx_hbm, outt_hbm, idx_v, buf_a, buf_b, buf_c,
                 outp_v, locb, bcolb, shared, sem_a, sem_b, sem_c):
    c = lax.axis_index("c")
    s = lax.axis_index("s")
    grp = c * 4 + s // 4
    cpos = s % 4
    f0 = pl.multiple_of(grp * _GF, _GF)

    pltpu.sync_copy(idx_hbm, idx_v)

    bufs = (buf_a, buf_b, buf_c)
    sems = (sem_a, sem_b, sem_c)

    def chunk_off(j):
        off = jnp.minimum((cpos * _CPT + j) * _W, _TAIL_OFF)
        return pl.multiple_of(off, 128)

    def fire(j):
        return pltpu.async_copy(
            memt_hbm.at[pl.ds(f0, _GF), pl.ds(chunk_off(j), _W)],
            bufs[j % _NBUF], sems[j % _NBUF])

    cps = [fire(0), fire(1)]
    lane = lax.iota(jnp.int32, _L)
    zeros_f = jnp.zeros((_L,), jnp.float32)
    zeros_i = jnp.zeros((_L,), jnp.int32)

    def zbody(vb, carry):
        for f in range(_GF):
            outp_v[f, pl.ds(vb * _L, _L)] = zeros_f
        return carry

    lax.fori_loop(0, BATCH // _L, zbody, 0)

    def scalar_of(vec):
        return jnp.sum(jnp.where(lane == 0, vec, 0))

    offs = [chunk_off(j) for j in range(_CPT)]

    def bbody(vb, carry):
        iv = idx_v[pl.ds(vb * _L, _L)]
        bcols = vb * _L + lane
        new_counts = []
        for j in range(_CPT):
            n_j = carry[j]
            loc = iv - offs[j]
            valid = jnp.logical_and(loc >= 0, loc < _W)
            plsc.store_compressed(locb.at[j, pl.ds(n_j, _L)], loc,
                                  mask=valid)
            plsc.store_compressed(bcolb.at[j, pl.ds(n_j, _L)], bcols,
                                  mask=valid)
            cnt = scalar_of(plsc.all_reduce_population_count(valid))
            new_counts.append(n_j + cnt)
        return tuple(new_counts)

    counts = lax.fori_loop(0, BATCH // _L, bbody, (0,) * _CPT)

    for j in range(_CPT):
        locb[j, pl.ds(counts[j], _L)] = zeros_i

    for j in range(_CPT):
        buf = bufs[j % _NBUF]
        cps[j].wait()
        if j + 2 < _CPT:
            cps.append(fire(j + 2))
        n_j = counts[j]
        nvec = jnp.full((_L,), n_j, jnp.int32)

        def gbody(i, carry, j=j, buf=buf, nvec=nvec):
            lv = locb[j, pl.ds(i * _L, _L)]
            bv = bcolb[j, pl.ds(i * _L, _L)]
            valid = (i * _L + lane) < nvec
            for f in range(_GF):
                vals = plsc.load_gather(
                    buf, [jnp.full((_L,), f, jnp.int32), lv])
                plsc.store_scatter(
                    outp_v, [jnp.full((_L,), f, jnp.int32), bv],
                    vals, mask=valid)
            return carry

        lax.fori_loop(0, (n_j + _L - 1) // _L, gbody, 0)

    pltpu.sync_copy(outp_v, shared.at[s])
    plsc.subcore_barrier()

    @pl.when(cpos == 0)
    def _merge_and_write():
        t1 = buf_a.at[:, pl.ds(0, BATCH)]
        t2 = buf_b.at[:, pl.ds(0, BATCH)]
        cp1 = pltpu.async_copy(shared.at[s + 1], t1, sem_a)
        cp2 = pltpu.async_copy(shared.at[s + 2], t2, sem_b)
        cp1.wait()
        cp2.wait()

        def mbody(vb, carry):
            for f in range(_GF):
                sl = pl.ds(vb * _L, _L)
                outp_v[f, sl] = outp_v[f, sl] + (buf_a[f, sl] + buf_b[f, sl])
            return carry

        lax.fori_loop(0, BATCH // _L, mbody, 0)
        cp3 = pltpu.async_copy(shared.at[s + 3], t1, sem_a)
        cp3.wait()

        def mbody2(vb, carry):
            for f in range(_GF):
                sl = pl.ds(vb * _L, _L)
                outp_v[f, sl] = outp_v[f, sl] + buf_a[f, sl]
            return carry

        lax.fori_loop(0, BATCH // _L, mbody2, 0)
        pltpu.sync_copy(outp_v, outt_hbm.at[pl.ds(f0, _GF), :])


_gather = functools.partial(
    pl.kernel,
    mesh=plsc.VectorSubcoreMesh(core_axis_name="c", subcore_axis_name="s"),
    out_type=jax.ShapeDtypeStruct((DIM, BATCH), jnp.float32),
    scratch_types=[
        pltpu.VMEM((BATCH,), jnp.int32),
        pltpu.VMEM((_GF, _W), jnp.float32),
        pltpu.VMEM((_GF, _W), jnp.float32),
        pltpu.VMEM((_GF, _W), jnp.float32),
        pltpu.VMEM((_GF, BATCH), jnp.float32),
        pltpu.VMEM((_CPT, BATCH + _L), jnp.int32),
        pltpu.VMEM((_CPT, BATCH + _L), jnp.int32),
        pltpu.VMEM_SHARED((_NS, _GF, BATCH), jnp.float32),
        pltpu.SemaphoreType.DMA,
        pltpu.SemaphoreType.DMA,
        pltpu.SemaphoreType.DMA,
    ],
    compiler_params=pltpu.CompilerParams(needs_layout_passes=False),
)(_gather_body)


def kernel(mem, target, idx):
    outt = _gather(mem.T, idx.astype(jnp.int32))
    return (outt.T, target)


# speedup vs baseline: 1.1167x; 1.1167x over previous
import functools

import jax
import jax.numpy as jnp
from jax import lax
from jax.experimental import pallas as pl
from jax.experimental.pallas import tpu as pltpu
from jax.experimental.pallas import tpu_sc as plsc

DS_SIZE = 100000
DIM = 64
BATCH = 1024

_info = plsc.get_sparse_core_info()
_NC, _NS, _L = _info.num_cores, _info.num_subcores, _info.num_lanes
_NGRP = 8
_GF = DIM // _NGRP
_CPT = 4
_W = 6272
_PAD_COLS = 100096
_TAIL_OFF = _PAD_COLS - _W


def _gather_body(memt_hbm, idx_hbm, outt_hbm, idx_v, buf_a, buf_b, outp_v,
                 locb, bcolb, shared, sem_a, sem_b):
    c = lax.axis_index("c")
    s = lax.axis_index("s")
    grp = c * 4 + s // 4
    cpos = s % 4
    f0 = pl.multiple_of(grp * _GF, _GF)

    pltpu.sync_copy(idx_hbm, idx_v)

    bufs = (buf_a, buf_b)
    sems = (sem_a, sem_b)

    def chunk_off(j):
        off = jnp.minimum((cpos * _CPT + j) * _W, _TAIL_OFF)
        return pl.multiple_of(off, 128)

    def fire(j):
        return pltpu.async_copy(
            memt_hbm.at[pl.ds(f0, _GF), pl.ds(chunk_off(j), _W)],
            bufs[j % 2], sems[j % 2])

    cp = fire(0)
    lane = lax.iota(jnp.int32, _L)
    zeros_f = jnp.zeros((_L,), jnp.float32)
    zeros_i = jnp.zeros((_L,), jnp.int32)

    def zbody(vb, carry):
        for f in range(_GF):
            outp_v[f, pl.ds(vb * _L, _L)] = zeros_f
        return carry

    lax.fori_loop(0, BATCH // _L, zbody, 0)

    def scalar_of(vec):
        return jnp.sum(jnp.where(lane == 0, vec, 0))

    offs = [chunk_off(j) for j in range(_CPT)]

    def bbody(vb, carry):
        iv = idx_v[pl.ds(vb * _L, _L)]
        bcols = vb * _L + lane
        new_counts = []
        for j in range(_CPT):
            n_j = carry[j]
            loc = iv - offs[j]
            valid = jnp.logical_and(loc >= 0, loc < _W)
            plsc.store_compressed(locb.at[j, pl.ds(n_j, _L)], loc,
                                  mask=valid)
            plsc.store_compressed(bcolb.at[j, pl.ds(n_j, _L)], bcols,
                                  mask=valid)
            cnt = scalar_of(plsc.all_reduce_population_count(valid))
            new_counts.append(n_j + cnt)
        return tuple(new_counts)

    counts = lax.fori_loop(0, BATCH // _L, bbody, (0, 0, 0, 0))

    for j in range(_CPT):
        locb[j, pl.ds(counts[j], _L)] = zeros_i

    for j in range(_CPT):
        buf = bufs[j % 2]
        cp.wait()
        if j + 1 < _CPT:
            cp = fire(j + 1)
        n_j = counts[j]
        nvec = jnp.full((_L,), n_j, jnp.int32)

        def gbody(i, carry, j=j, buf=buf, nvec=nvec):
            lv = locb[j, pl.ds(i * _L, _L)]
            bv = bcolb[j, pl.ds(i * _L, _L)]
            valid = (i * _L + lane) < nvec
            for f in range(_GF):
                vals = plsc.load_gather(
                    buf, [jnp.full((_L,), f, jnp.int32), lv])
                plsc.store_scatter(
                    outp_v, [jnp.full((_L,), f, jnp.int32), bv],
                    vals, mask=valid)
            return carry

        lax.fori_loop(0, (n_j + _L - 1) // _L, gbody, 0)

    pltpu.sync_copy(outp_v, shared.at[s])
    plsc.subcore_barrier()

    quarter = BATCH // 4
    q0 = pl.multiple_of(cpos * quarter, quarter)
    sbase = s - cpos
    temps = [buf_a.at[:, pl.ds(k * quarter, quarter)] for k in range(3)]
    tsems = (sem_a, sem_b, sem_a)
    cps2 = []
    for k in range(3):
        slot = sbase + lax.rem(cpos + k + 1, 4)
        cps2.append(pltpu.async_copy(
            shared.at[slot, :, pl.ds(q0, quarter)], temps[k], tsems[k]))
    for cp2 in cps2:
        cp2.wait()

    def mbody(i, carry):
        for f in range(_GF):
            sl = pl.ds(q0 + i * _L, _L)
            acc = outp_v[f, sl] + buf_a[f, pl.ds(i * _L, _L)]
            acc = acc + (buf_a[f, pl.ds(quarter + i * _L, _L)]
                         + buf_a[f, pl.ds(2 * quarter + i * _L, _L)])
            outp_v[f, sl] = acc
        return carry

    lax.fori_loop(0, quarter // _L, mbody, 0)
    pltpu.sync_copy(outp_v.at[:, pl.ds(q0, quarter)],
                    outt_hbm.at[pl.ds(f0, _GF), pl.ds(q0, quarter)])


_gather = functools.partial(
    pl.kernel,
    mesh=plsc.VectorSubcoreMesh(core_axis_name="c", subcore_axis_name="s"),
    out_type=jax.ShapeDtypeStruct((DIM, BATCH), jnp.float32),
    scratch_types=[
        pltpu.VMEM((BATCH,), jnp.int32),
        pltpu.VMEM((_GF, _W), jnp.float32),
        pltpu.VMEM((_GF, _W), jnp.float32),
        pltpu.VMEM((_GF, BATCH), jnp.float32),
        pltpu.VMEM((_CPT, BATCH + _L), jnp.int32),
        pltpu.VMEM((_CPT, BATCH + _L), jnp.int32),
        pltpu.VMEM_SHARED((_NS, _GF, BATCH), jnp.float32),
        pltpu.SemaphoreType.DMA,
        pltpu.SemaphoreType.DMA,
    ],
    compiler_params=pltpu.CompilerParams(needs_layout_passes=False),
)(_gather_body)


def kernel(mem, target, idx):
    outt = _gather(mem.T, idx.astype(jnp.int32))
    return (outt.T, target)
